# baseline (speedup 1.0000x reference)
import functools

import jax
import jax.numpy as jnp
import numpy as np
from jax import lax
from jax.experimental import pallas as pl
from jax.experimental.pallas import tpu as pltpu
from jax.experimental.pallas import tpu_sc as plsc

_DOMAIN_TO_CHOSEN = ((0, 2, 5), (1, 3, 6), (2, 4, 7), (0, 1, 2))

_NC, _NS = 1, 16
_NW = _NC * _NS
_CR = 8
_LANES = 16

_N_ROWS = 4096
_N_COLS = 2048
_ROWS_TC = 3072

_ROWS_PER_BLOCK = 256


def _tc_body(di_ref, tab_ref, x0_ref, x1_ref, x2_ref, o_ref):
    del di_ref, tab_ref
    o_ref[...] = (x0_ref[0] + x1_ref[0] + x2_ref[0]) * jnp.float32(1.0 / 3.0)


def _tc_mean(inputs, di_arr, table, n_rows, out_rows):
    nb = n_rows // _ROWS_PER_BLOCK
    blk = (1, _ROWS_PER_BLOCK, _N_COLS)

    def in_spec(k):
        return pl.BlockSpec(
            blk, lambda i, di_ref, tab_ref, _k=k: (tab_ref[di_ref[0], _k], i, 0))

    grid_spec = pltpu.PrefetchScalarGridSpec(
        num_scalar_prefetch=2,
        grid=(nb,),
        in_specs=[in_spec(0), in_spec(1), in_spec(2)],
        out_specs=pl.BlockSpec((_ROWS_PER_BLOCK, _N_COLS),
                               lambda i, di_ref, tab_ref: (i, 0)),
    )
    return pl.pallas_call(
        _tc_body,
        grid_spec=grid_spec,
        out_shape=jax.ShapeDtypeStruct((out_rows, _N_COLS), inputs.dtype),
    )(di_arr, table, inputs, inputs, inputs)


def _sc_mean(inputs_flat, idx_flat, n_rows):
    n_chunks = n_rows // _CR
    cpw = n_chunks // _NW
    iw = 3 * _CR
    mesh = plsc.VectorSubcoreMesh(
        core_axis_name="c", subcore_axis_name="s",
        num_cores=_NC, num_subcores=_NS)

    @functools.partial(
        pl.kernel,
        mesh=mesh,
        out_type=jax.ShapeDtypeStruct((n_rows, _N_COLS), jnp.float32),
        scratch_types=[
            pltpu.VMEM((cpw * iw,), jnp.int32),
            pltpu.VMEM((3 * _CR, _N_COLS), jnp.float32),
            pltpu.VMEM((3 * _CR, _N_COLS), jnp.float32),
            pltpu.VMEM((_CR, _N_COLS), jnp.float32),
            pltpu.SemaphoreType.DMA,
            pltpu.SemaphoreType.DMA,
            pltpu.SemaphoreType.DMA,
        ],
    )
    def sc_k(in_hbm, idx_hbm, out_hbm, idx_all, rows0, rows1,
             obuf, gsem0, gsem1, osem):
        wid = lax.axis_index("s") * _NC + lax.axis_index("c")
        base_chunk = wid * cpw
        rows = (rows0, rows1)
        gsems = (gsem0, gsem1)

        pltpu.sync_copy(idx_hbm.at[pl.ds(base_chunk * iw, cpw * iw)], idx_all)

        def gather(c, b):
            return pltpu.make_async_copy(
                in_hbm.at[idx_all.at[pl.ds(c * iw, iw)]], rows[b], gsems[b])

        def out_copy(c):
            return pltpu.make_async_copy(
                obuf, out_hbm.at[pl.ds((base_chunk + c) * _CR, _CR)], osem)

        def mean_chunk(r_v):
            @plsc.parallel_loop(0, _N_COLS, _LANES, unroll=4)
            def _(i):
                s = pl.ds(i, _LANES)
                for r in range(_CR):
                    m = r_v[r, s] + r_v[r + _CR, s] + r_v[r + 2 * _CR, s]
                    obuf[r, s] = m * jnp.float32(1.0 / 3.0)

        gather(0, 0).start()

        if cpw == 1:
            gather(0, 0).wait()
            mean_chunk(rows0)
            out_copy(0).start()
            out_copy(0).wait()
            return

        def pair_body(g, carry):
            for b in range(2):
                c = 2 * g + b
                gather(c, b).wait()
                if b == 0:
                    gather(c + 1, 1).start()
                else:
                    @pl.when(g < cpw // 2 - 1)
                    def _():
                        gather(c + 1, 0).start()
                if b == 0:
                    @pl.when(g > 0)
                    def _():
                        out_copy(c - 1).wait()
                else:
                    out_copy(c - 1).wait()

                mean_chunk(rows[b])
                out_copy(c).start()
            return carry

        lax.fori_loop(0, cpw // 2, pair_body, 0)
        out_copy(cpw - 1).wait()

    return sc_k(inputs_flat, idx_flat)


def kernel(inputs, domain_idx):
    n_cand, n_rows, n_cols = inputs.shape
    di_arr = jnp.asarray(domain_idx, dtype=jnp.int32).reshape(1)
    table = jnp.asarray(_DOMAIN_TO_CHOSEN, dtype=jnp.int32)

    n_sc = n_rows - _ROWS_TC
    if n_sc == 0:
        return _tc_mean(inputs, di_arr, table, n_rows, n_rows)

    inputs_flat = inputs.reshape(n_cand * n_rows, n_cols)
    np_tab = np.asarray(_DOMAIN_TO_CHOSEN, dtype=np.int32)
    np_rows = np.arange(_ROWS_TC, n_rows, dtype=np.int32).reshape(
        n_sc // _CR, 1, _CR)
    np_idx = (np_tab[:, None, :, None] * n_rows +
              np_rows[None]).reshape(4, -1)
    idx_flat = jnp.asarray(np_idx)[domain_idx]
    sc_part = _sc_mean(inputs_flat, idx_flat, n_sc)
    if _ROWS_TC == 0:
        return sc_part
    tc_full = _tc_mean(inputs, di_arr, table, _ROWS_TC, n_rows)
    return lax.dynamic_update_slice(tc_full, sc_part, (_ROWS_TC, 0))


# speedup vs baseline: 1.0915x; 1.0915x over previous
import functools

import jax
import jax.numpy as jnp
import numpy as np
from jax import lax
from jax.experimental import pallas as pl
from jax.experimental.pallas import tpu as pltpu
from jax.experimental.pallas import tpu_sc as plsc

_DOMAIN_TO_CHOSEN = ((0, 2, 5), (1, 3, 6), (2, 4, 7), (0, 1, 2))

_NC, _NS = 1, 16
_NW = _NC * _NS
_CR = 8
_LANES = 16

_N_ROWS = 4096
_N_COLS = 2048
_ROWS_TC = 3840

_ROWS_PER_BLOCK = 256


def _tc_body(di_ref, tab_ref, x0_ref, x1_ref, x2_ref, o_ref):
    del di_ref, tab_ref
    o_ref[...] = (x0_ref[0] + x1_ref[0] + x2_ref[0]) * jnp.float32(1.0 / 3.0)


def _tc_mean(inputs, di_arr, table, n_rows, out_rows):
    nb = n_rows // _ROWS_PER_BLOCK
    blk = (1, _ROWS_PER_BLOCK, _N_COLS)

    def in_spec(k):
        return pl.BlockSpec(
            blk, lambda i, di_ref, tab_ref, _k=k: (tab_ref[di_ref[0], _k], i, 0))

    grid_spec = pltpu.PrefetchScalarGridSpec(
        num_scalar_prefetch=2,
        grid=(nb,),
        in_specs=[in_spec(0), in_spec(1), in_spec(2)],
        out_specs=pl.BlockSpec((_ROWS_PER_BLOCK, _N_COLS),
                               lambda i, di_ref, tab_ref: (i, 0)),
    )
    return pl.pallas_call(
        _tc_body,
        grid_spec=grid_spec,
        out_shape=jax.ShapeDtypeStruct((out_rows, _N_COLS), inputs.dtype),
    )(di_arr, table, inputs, inputs, inputs)


def _sc_mean(inputs_flat, idx_flat, n_rows):
    n_chunks = n_rows // _CR
    cpw = n_chunks // _NW
    iw = 3 * _CR
    mesh = plsc.VectorSubcoreMesh(
        core_axis_name="c", subcore_axis_name="s",
        num_cores=_NC, num_subcores=_NS)

    @functools.partial(
        pl.kernel,
        mesh=mesh,
        out_type=jax.ShapeDtypeStruct((n_rows, _N_COLS), jnp.float32),
        scratch_types=[
            pltpu.VMEM((cpw * iw,), jnp.int32),
            pltpu.VMEM((3 * _CR, _N_COLS), jnp.float32),
            pltpu.VMEM((3 * _CR, _N_COLS), jnp.float32),
            pltpu.VMEM((_CR, _N_COLS), jnp.float32),
            pltpu.SemaphoreType.DMA,
            pltpu.SemaphoreType.DMA,
            pltpu.SemaphoreType.DMA,
        ],
    )
    def sc_k(in_hbm, idx_hbm, out_hbm, idx_all, rows0, rows1,
             obuf, gsem0, gsem1, osem):
        wid = lax.axis_index("s") * _NC + lax.axis_index("c")
        base_chunk = wid * cpw
        rows = (rows0, rows1)
        gsems = (gsem0, gsem1)

        pltpu.sync_copy(idx_hbm.at[pl.ds(base_chunk * iw, cpw * iw)], idx_all)

        def gather(c, b):
            return pltpu.make_async_copy(
                in_hbm.at[idx_all.at[pl.ds(c * iw, iw)]], rows[b], gsems[b])

        def out_copy(c):
            return pltpu.make_async_copy(
                obuf, out_hbm.at[pl.ds((base_chunk + c) * _CR, _CR)], osem)

        def mean_chunk(r_v):
            @plsc.parallel_loop(0, _N_COLS, _LANES, unroll=4)
            def _(i):
                s = pl.ds(i, _LANES)
                for r in range(_CR):
                    m = r_v[r, s] + r_v[r + _CR, s] + r_v[r + 2 * _CR, s]
                    obuf[r, s] = m * jnp.float32(1.0 / 3.0)

        gather(0, 0).start()

        if cpw == 1:
            gather(0, 0).wait()
            mean_chunk(rows0)
            out_copy(0).start()
            out_copy(0).wait()
            return

        def pair_body(g, carry):
            for b in range(2):
                c = 2 * g + b
                gather(c, b).wait()
                if b == 0:
                    gather(c + 1, 1).start()
                else:
                    @pl.when(g < cpw // 2 - 1)
                    def _():
                        gather(c + 1, 0).start()
                if b == 0:
                    @pl.when(g > 0)
                    def _():
                        out_copy(c - 1).wait()
                else:
                    out_copy(c - 1).wait()

                mean_chunk(rows[b])
                out_copy(c).start()
            return carry

        lax.fori_loop(0, cpw // 2, pair_body, 0)
        out_copy(cpw - 1).wait()

    return sc_k(inputs_flat, idx_flat)


def kernel(inputs, domain_idx):
    n_cand, n_rows, n_cols = inputs.shape
    di_arr = jnp.asarray(domain_idx, dtype=jnp.int32).reshape(1)
    table = jnp.asarray(_DOMAIN_TO_CHOSEN, dtype=jnp.int32)

    n_sc = n_rows - _ROWS_TC
    if n_sc == 0:
        return _tc_mean(inputs, di_arr, table, n_rows, n_rows)

    inputs_flat = inputs.reshape(n_cand * n_rows, n_cols)
    np_tab = np.asarray(_DOMAIN_TO_CHOSEN, dtype=np.int32)
    np_rows = np.arange(_ROWS_TC, n_rows, dtype=np.int32).reshape(
        n_sc // _CR, 1, _CR)
    np_idx = (np_tab[:, None, :, None] * n_rows +
              np_rows[None]).reshape(4, -1)
    idx_flat = jnp.asarray(np_idx)[domain_idx]
    sc_part = _sc_mean(inputs_flat, idx_flat, n_sc)
    if _ROWS_TC == 0:
        return sc_part
    tc_full = _tc_mean(inputs, di_arr, table, _ROWS_TC, n_rows)
    return lax.dynamic_update_slice(tc_full, sc_part, (_ROWS_TC, 0))


# speedup vs baseline: 1.1308x; 1.0360x over previous
import functools

import jax
import jax.numpy as jnp
import numpy as np
from jax import lax
from jax.experimental import pallas as pl
from jax.experimental.pallas import tpu as pltpu
from jax.experimental.pallas import tpu_sc as plsc

_DOMAIN_TO_CHOSEN = ((0, 2, 5), (1, 3, 6), (2, 4, 7), (0, 1, 2))

_NC, _NS = 1, 16
_NW = _NC * _NS
_CR = 8
_LANES = 16

_N_ROWS = 4096
_N_COLS = 2048
_ROWS_TC = 3968

_ROWS_PER_BLOCK = 256


def _tc_body(di_ref, tab_ref, x0_ref, x1_ref, x2_ref, o_ref):
    del di_ref, tab_ref
    o_ref[...] = (x0_ref[0] + x1_ref[0] + x2_ref[0]) * jnp.float32(1.0 / 3.0)


def _tc_mean(inputs, di_arr, table, n_rows, out_rows):
    nb = n_rows // _ROWS_PER_BLOCK
    blk = (1, _ROWS_PER_BLOCK, _N_COLS)

    def in_spec(k):
        return pl.BlockSpec(
            blk, lambda i, di_ref, tab_ref, _k=k: (tab_ref[di_ref[0], _k], i, 0))

    grid_spec = pltpu.PrefetchScalarGridSpec(
        num_scalar_prefetch=2,
        grid=(nb,),
        in_specs=[in_spec(0), in_spec(1), in_spec(2)],
        out_specs=pl.BlockSpec((_ROWS_PER_BLOCK, _N_COLS),
                               lambda i, di_ref, tab_ref: (i, 0)),
    )
    return pl.pallas_call(
        _tc_body,
        grid_spec=grid_spec,
        out_shape=jax.ShapeDtypeStruct((out_rows, _N_COLS), inputs.dtype),
    )(di_arr, table, inputs, inputs, inputs)


def _sc_mean(inputs_flat, idx_flat, n_rows):
    n_chunks = n_rows // _CR
    cpw = n_chunks // _NW
    iw = 3 * _CR
    mesh = plsc.VectorSubcoreMesh(
        core_axis_name="c", subcore_axis_name="s",
        num_cores=_NC, num_subcores=_NS)

    @functools.partial(
        pl.kernel,
        mesh=mesh,
        out_type=jax.ShapeDtypeStruct((n_rows, _N_COLS), jnp.float32),
        scratch_types=[
            pltpu.VMEM((cpw * iw,), jnp.int32),
            pltpu.VMEM((3 * _CR, _N_COLS), jnp.float32),
            pltpu.VMEM((3 * _CR, _N_COLS), jnp.float32),
            pltpu.VMEM((_CR, _N_COLS), jnp.float32),
            pltpu.SemaphoreType.DMA,
            pltpu.SemaphoreType.DMA,
            pltpu.SemaphoreType.DMA,
        ],
    )
    def sc_k(in_hbm, idx_hbm, out_hbm, idx_all, rows0, rows1,
             obuf, gsem0, gsem1, osem):
        wid = lax.axis_index("s") * _NC + lax.axis_index("c")
        base_chunk = wid * cpw
        rows = (rows0, rows1)
        gsems = (gsem0, gsem1)

        pltpu.sync_copy(idx_hbm.at[pl.ds(base_chunk * iw, cpw * iw)], idx_all)

        def gather(c, b):
            return pltpu.make_async_copy(
                in_hbm.at[idx_all.at[pl.ds(c * iw, iw)]], rows[b], gsems[b])

        def out_copy(c):
            return pltpu.make_async_copy(
                obuf, out_hbm.at[pl.ds((base_chunk + c) * _CR, _CR)], osem)

        def mean_chunk(r_v):
            @plsc.parallel_loop(0, _N_COLS, _LANES, unroll=4)
            def _(i):
                s = pl.ds(i, _LANES)
                for r in range(_CR):
                    m = r_v[r, s] + r_v[r + _CR, s] + r_v[r + 2 * _CR, s]
                    obuf[r, s] = m * jnp.float32(1.0 / 3.0)

        gather(0, 0).start()

        if cpw == 1:
            gather(0, 0).wait()
            mean_chunk(rows0)
            out_copy(0).start()
            out_copy(0).wait()
            return

        def pair_body(g, carry):
            for b in range(2):
                c = 2 * g + b
                gather(c, b).wait()
                if b == 0:
                    gather(c + 1, 1).start()
                else:
                    @pl.when(g < cpw // 2 - 1)
                    def _():
                        gather(c + 1, 0).start()
                if b == 0:
                    @pl.when(g > 0)
                    def _():
                        out_copy(c - 1).wait()
                else:
                    out_copy(c - 1).wait()

                mean_chunk(rows[b])
                out_copy(c).start()
            return carry

        lax.fori_loop(0, cpw // 2, pair_body, 0)
        out_copy(cpw - 1).wait()

    return sc_k(inputs_flat, idx_flat)


def kernel(inputs, domain_idx):
    n_cand, n_rows, n_cols = inputs.shape
    di_arr = jnp.asarray(domain_idx, dtype=jnp.int32).reshape(1)
    table = jnp.asarray(_DOMAIN_TO_CHOSEN, dtype=jnp.int32)

    n_sc = n_rows - _ROWS_TC
    if n_sc == 0:
        return _tc_mean(inputs, di_arr, table, n_rows, n_rows)

    inputs_flat = inputs.reshape(n_cand * n_rows, n_cols)
    np_tab = np.asarray(_DOMAIN_TO_CHOSEN, dtype=np.int32)
    np_rows = np.arange(_ROWS_TC, n_rows, dtype=np.int32).reshape(
        n_sc // _CR, 1, _CR)
    np_idx = (np_tab[:, None, :, None] * n_rows +
              np_rows[None]).reshape(4, -1)
    idx_flat = jnp.asarray(np_idx)[domain_idx]
    sc_part = _sc_mean(inputs_flat, idx_flat, n_sc)
    if _ROWS_TC == 0:
        return sc_part
    tc_full = _tc_mean(inputs, di_arr, table, _ROWS_TC, n_rows)
    return lax.dynamic_update_slice(tc_full, sc_part, (_ROWS_TC, 0))
